# baseline (speedup 1.0000x reference)
import jax
import jax.numpy as jnp
from jax.experimental import pallas as pl
from jax.experimental.pallas import tpu as pltpu

_SIZE = 1000
_NCLS = 10
_NPTS = 8
_TOT = _SIZE * _SIZE * _NCLS
_LANES = 128
_ROWS = _TOT // _LANES
_BRR = 8000
_NB = (_ROWS + _BRR - 1) // _BRR


def _viz_kernel(z_ref, lab_ref, out_ref):
    blk = pl.program_id(0)
    row0 = blk * _BRR
    out_ref[...] = jnp.zeros_like(out_ref)
    lane_ids = jax.lax.broadcasted_iota(jnp.int32, (1, _LANES), 1)
    for n in range(_NPTS):
        a = z_ref[n, 0] * _SIZE
        b = z_ref[n, 1] * _SIZE
        j = a.astype(jnp.int32)
        i = b.astype(jnp.int32)
        valid = (a > j.astype(jnp.float32)) & (b > i.astype(jnp.float32))
        lin = (j * (_SIZE * _NCLS) + i * _NCLS + lab_ref[n])
        r = lin // _LANES
        l = lin - r * _LANES
        in_block = valid & (r >= row0) & (r < row0 + _BRR)

        @pl.when(in_block)
        def _():
            rl = r - row0
            row = (lane_ids == l).astype(jnp.float32)
            cur = out_ref[pl.ds(rl, 1), :]
            out_ref[pl.ds(rl, 1), :] = jnp.maximum(cur, row)


def kernel(z, labels):
    labels = labels.astype(jnp.int32)
    out = pl.pallas_call(
        _viz_kernel,
        out_shape=jax.ShapeDtypeStruct((_ROWS, _LANES), jnp.float32),
        grid=(_NB,),
        in_specs=[
            pl.BlockSpec(memory_space=pltpu.SMEM),
            pl.BlockSpec(memory_space=pltpu.SMEM),
        ],
        out_specs=pl.BlockSpec((_BRR, _LANES), lambda m: (m, 0)),
        compiler_params=pltpu.CompilerParams(
            dimension_semantics=("parallel",),
        ),
        name="visual_imitation",
    )(z, labels)
    return out.reshape(_SIZE, _SIZE, _NCLS)


# speedup vs baseline: 44.1376x; 44.1376x over previous
import jax
import jax.numpy as jnp
from jax.experimental import pallas as pl
from jax.experimental.pallas import tpu as pltpu

_SIZE = 1000
_NCLS = 10
_NPTS = 8
_BJ = 200
_NB = _SIZE // _BJ


def _viz_kernel(z_ref, lab_ref, out_ref):
    blk = pl.program_id(0)
    j0 = blk * _BJ
    out_ref[...] = jnp.zeros_like(out_ref)
    i_ids = jax.lax.broadcasted_iota(jnp.int32, (1, _SIZE), 1)
    for n in range(_NPTS):
        a = z_ref[n, 0] * _SIZE
        b = z_ref[n, 1] * _SIZE
        j = a.astype(jnp.int32)
        i = b.astype(jnp.int32)
        valid = (a > j.astype(jnp.float32)) & (b > i.astype(jnp.float32))
        in_block = valid & (j >= j0) & (j < j0 + _BJ)

        @pl.when(in_block)
        def _():
            jl = j - j0
            row = (i_ids == i).astype(jnp.float32)
            cur = out_ref[lab_ref[n], pl.ds(jl, 1), :]
            out_ref[lab_ref[n], pl.ds(jl, 1), :] = jnp.maximum(cur, row)


def kernel(z, labels):
    labels = labels.astype(jnp.int32)
    out = pl.pallas_call(
        _viz_kernel,
        out_shape=jax.ShapeDtypeStruct((_NCLS, _SIZE, _SIZE), jnp.float32),
        grid=(_NB,),
        in_specs=[
            pl.BlockSpec(memory_space=pltpu.SMEM),
            pl.BlockSpec(memory_space=pltpu.SMEM),
        ],
        out_specs=pl.BlockSpec((_NCLS, _BJ, _SIZE), lambda m: (0, m, 0)),
        compiler_params=pltpu.CompilerParams(
            dimension_semantics=("parallel",),
        ),
        name="visual_imitation",
    )(z, labels)
    return jnp.transpose(out, (1, 2, 0))
